# baseline (speedup 1.0000x reference)
import functools

import jax
import jax.numpy as jnp
from jax import lax
from jax.experimental import pallas as pl
from jax.experimental.pallas import tpu as pltpu
from jax.experimental.pallas import tpu_sc as plsc

_M = 256
_WINDOWS = ((0, 2816), (2816, 5632), (5632, 8192))


def _vq_argmin_body(z_ref, cb_ref, idx_ref, c2_ref):
    i = pl.program_id(0)
    M, D = z_ref.shape
    K = cb_ref.shape[0]

    @pl.when(i == 0)
    def _():
        cb = cb_ref[...]
        c2_ref[...] = jnp.sum(cb * cb, axis=1).reshape(1, K)

    z = z_ref[...]
    z2 = jnp.sum(z * z, axis=1, keepdims=True)
    zb = z.astype(jnp.bfloat16)

    best_v = jnp.full((M, 1), jnp.inf, dtype=jnp.float32)
    best_i = jnp.zeros((M, 1), dtype=jnp.int32)
    for lo, hi in _WINDOWS:
        W = hi - lo
        cbc = cb_ref[pl.ds(lo, W), :]
        cross = lax.dot_general(zb, cbc.astype(jnp.bfloat16),
                                (((1,), (1,)), ((), ())),
                                preferred_element_type=jnp.float32)
        c2c = c2_ref[:, pl.ds(lo, W)]
        dist = jnp.sqrt(jnp.maximum((z2 + c2c) - 2.0 * cross, 0.0))
        wv = jnp.min(dist, axis=1, keepdims=True)
        ids = lax.broadcasted_iota(jnp.int32, (M, W), 1)
        wi = jnp.min(jnp.where(dist == wv, ids, K), axis=1,
                     keepdims=True) + lo
        upd = wv < best_v
        best_i = jnp.where(upd, wi, best_i)
        best_v = jnp.where(upd, wv, best_v).astype(jnp.bfloat16).astype(
            jnp.float32)
    idx_ref[0, 0, :] = best_i.reshape(M)


def _argmin_indices(z2d, codebook):
    BT, D = z2d.shape
    K = codebook.shape[0]
    n_t = BT // _M
    idx3 = pl.pallas_call(
        _vq_argmin_body,
        grid=(n_t,),
        in_specs=[
            pl.BlockSpec((_M, D), lambda i: (i, 0)),
            pl.BlockSpec((K, D), lambda i: (0, 0)),
        ],
        out_specs=pl.BlockSpec((1, 1, _M), lambda i: (i, 0, 0)),
        out_shape=jax.ShapeDtypeStruct((n_t, 1, _M), jnp.int32),
        scratch_shapes=[pltpu.VMEM((1, K), jnp.float32)],
        compiler_params=pltpu.CompilerParams(
            dimension_semantics=("arbitrary",)),
    )(z2d, codebook)
    return idx3.reshape(BT)


@functools.lru_cache(maxsize=None)
def _make_gather(BT, D, K):
    info = plsc.get_sparse_core_info()
    NC, NS = info.num_cores, info.num_subcores
    NW = NC * NS
    b_per_w = BT // NW
    CH = 128
    n_ch = b_per_w // CH
    mesh = plsc.VectorSubcoreMesh(core_axis_name="c", subcore_axis_name="s")

    @functools.partial(
        pl.kernel,
        mesh=mesh,
        out_type=jax.ShapeDtypeStruct((BT, D), jnp.float32),
        scratch_types=[
            pltpu.VMEM((CH,), jnp.int32),
            pltpu.VMEM((CH, D), jnp.float32),
            pltpu.SemaphoreType.DMA,
        ],
    )
    def gather_k(idx_hbm, cb_hbm, out_hbm, idx_v, rows_v, sem):
        wid = lax.axis_index("s") * NC + lax.axis_index("c")
        for c in range(n_ch):
            base = wid * b_per_w + c * CH
            pltpu.sync_copy(idx_hbm.at[pl.ds(base, CH)], idx_v)
            pltpu.async_copy(cb_hbm.at[idx_v], rows_v, sem).wait()
            pltpu.sync_copy(rows_v, out_hbm.at[pl.ds(base, CH)])

    return gather_k


def kernel(z_continuous, codebook):
    B, T, D = z_continuous.shape
    K = codebook.shape[0]
    BT = B * T
    z2d = z_continuous.reshape(BT, D)
    idx = _argmin_indices(z2d, codebook)
    z_q = _make_gather(BT, D, K)(idx, codebook)
    return z_q.reshape(B, T, D)


# speedup vs baseline: 1.0580x; 1.0580x over previous
import functools

import jax
import jax.numpy as jnp
from jax import lax
from jax.experimental import pallas as pl
from jax.experimental.pallas import tpu as pltpu
from jax.experimental.pallas import tpu_sc as plsc

_M = 512
_WINDOWS = ((0, 2816), (2816, 5632), (5632, 8192))


def _vq_argmin_body(z_ref, cb_ref, idx_ref, c2_ref):
    i = pl.program_id(0)
    M, D = z_ref.shape
    K = cb_ref.shape[0]

    @pl.when(i == 0)
    def _():
        cb = cb_ref[...]
        c2_ref[...] = jnp.sum(cb * cb, axis=1).reshape(1, K)

    z = z_ref[...]
    z2 = jnp.sum(z * z, axis=1, keepdims=True)
    zb = z.astype(jnp.bfloat16)

    best_v = jnp.full((M, 1), jnp.inf, dtype=jnp.float32)
    best_i = jnp.zeros((M, 1), dtype=jnp.int32)
    for lo, hi in _WINDOWS:
        W = hi - lo
        cbc = cb_ref[pl.ds(lo, W), :]
        cross = lax.dot_general(zb, cbc.astype(jnp.bfloat16),
                                (((1,), (1,)), ((), ())),
                                preferred_element_type=jnp.float32)
        c2c = c2_ref[:, pl.ds(lo, W)]
        dist = jnp.sqrt(jnp.maximum((z2 + c2c) - 2.0 * cross, 0.0))
        wv = jnp.min(dist, axis=1, keepdims=True)
        ids = lax.broadcasted_iota(jnp.int32, (M, W), 1)
        wi = jnp.min(jnp.where(dist == wv, ids, K), axis=1,
                     keepdims=True) + lo
        upd = wv < best_v
        best_i = jnp.where(upd, wi, best_i)
        best_v = jnp.where(upd, wv, best_v).astype(jnp.bfloat16).astype(
            jnp.float32)
    idx_ref[0, 0, :] = best_i.reshape(M)


def _argmin_indices(z2d, codebook):
    BT, D = z2d.shape
    K = codebook.shape[0]
    n_t = BT // _M
    idx3 = pl.pallas_call(
        _vq_argmin_body,
        grid=(n_t,),
        in_specs=[
            pl.BlockSpec((_M, D), lambda i: (i, 0)),
            pl.BlockSpec((K, D), lambda i: (0, 0)),
        ],
        out_specs=pl.BlockSpec((1, 1, _M), lambda i: (i, 0, 0)),
        out_shape=jax.ShapeDtypeStruct((n_t, 1, _M), jnp.int32),
        scratch_shapes=[pltpu.VMEM((1, K), jnp.float32)],
        compiler_params=pltpu.CompilerParams(
            dimension_semantics=("arbitrary",)),
    )(z2d, codebook)
    return idx3.reshape(BT)


@functools.lru_cache(maxsize=None)
def _make_gather(BT, D, K):
    info = plsc.get_sparse_core_info()
    NC, NS = info.num_cores, info.num_subcores
    NW = NC * NS
    b_per_w = BT // NW
    CH = 128
    n_ch = b_per_w // CH
    mesh = plsc.VectorSubcoreMesh(core_axis_name="c", subcore_axis_name="s")

    @functools.partial(
        pl.kernel,
        mesh=mesh,
        out_type=jax.ShapeDtypeStruct((BT, D), jnp.float32),
        scratch_types=[
            pltpu.VMEM((CH,), jnp.int32),
            pltpu.VMEM((CH, D), jnp.float32),
            pltpu.SemaphoreType.DMA,
        ],
    )
    def gather_k(idx_hbm, cb_hbm, out_hbm, idx_v, rows_v, sem):
        wid = lax.axis_index("s") * NC + lax.axis_index("c")
        for c in range(n_ch):
            base = wid * b_per_w + c * CH
            pltpu.sync_copy(idx_hbm.at[pl.ds(base, CH)], idx_v)
            pltpu.async_copy(cb_hbm.at[idx_v], rows_v, sem).wait()
            pltpu.sync_copy(rows_v, out_hbm.at[pl.ds(base, CH)])

    return gather_k


def kernel(z_continuous, codebook):
    B, T, D = z_continuous.shape
    K = codebook.shape[0]
    BT = B * T
    z2d = z_continuous.reshape(BT, D)
    idx = _argmin_indices(z2d, codebook)
    z_q = _make_gather(BT, D, K)(idx, codebook)
    return z_q.reshape(B, T, D)


# speedup vs baseline: 1.1083x; 1.0476x over previous
import functools

import jax
import jax.numpy as jnp
from jax import lax
from jax.experimental import pallas as pl
from jax.experimental.pallas import tpu as pltpu
from jax.experimental.pallas import tpu_sc as plsc

_M = 512
_WINDOWS = ((0, 2816), (2816, 5632), (5632, 8192))


def _vq_argmin_body(z_ref, cb_ref, idx_ref, c2_ref, cbb_ref):
    i = pl.program_id(0)
    M, D = z_ref.shape
    K = cb_ref.shape[0]

    @pl.when(i == 0)
    def _():
        cb = cb_ref[...]
        c2_ref[...] = jnp.sum(cb * cb, axis=1).reshape(1, K)
        cbb_ref[...] = cb.astype(jnp.bfloat16)

    z = z_ref[...]
    z2 = jnp.sum(z * z, axis=1, keepdims=True)
    zb = (-2.0 * z).astype(jnp.bfloat16)

    best_v = jnp.full((M, 1), jnp.inf, dtype=jnp.float32)
    best_i = jnp.zeros((M, 1), dtype=jnp.int32)
    for lo, hi in _WINDOWS:
        W = hi - lo
        ncross = lax.dot_general(zb, cbb_ref[pl.ds(lo, W), :],
                                 (((1,), (1,)), ((), ())),
                                 preferred_element_type=jnp.float32)
        c2c = c2_ref[:, pl.ds(lo, W)]
        dist = jnp.sqrt(jnp.maximum((z2 + c2c) + ncross, 0.0))
        wv = jnp.min(dist, axis=1, keepdims=True)
        ids = lax.broadcasted_iota(jnp.int32, (M, W), 1)
        wi = jnp.min(jnp.where(dist == wv, ids, K), axis=1,
                     keepdims=True) + lo
        upd = wv < best_v
        best_i = jnp.where(upd, wi, best_i)
        best_v = jnp.where(upd, wv, best_v).astype(jnp.bfloat16).astype(
            jnp.float32)
    idx_ref[0, 0, :] = best_i.reshape(M)


def _argmin_indices(z2d, codebook):
    BT, D = z2d.shape
    K = codebook.shape[0]
    n_t = BT // _M
    idx3 = pl.pallas_call(
        _vq_argmin_body,
        grid=(n_t,),
        in_specs=[
            pl.BlockSpec((_M, D), lambda i: (i, 0)),
            pl.BlockSpec((K, D), lambda i: (0, 0)),
        ],
        out_specs=pl.BlockSpec((1, 1, _M), lambda i: (i, 0, 0)),
        out_shape=jax.ShapeDtypeStruct((n_t, 1, _M), jnp.int32),
        scratch_shapes=[pltpu.VMEM((1, K), jnp.float32),
                        pltpu.VMEM((K, D), jnp.bfloat16)],
        compiler_params=pltpu.CompilerParams(
            dimension_semantics=("arbitrary",)),
    )(z2d, codebook)
    return idx3.reshape(BT)


@functools.lru_cache(maxsize=None)
def _make_gather(BT, D, K):
    info = plsc.get_sparse_core_info()
    NC, NS = info.num_cores, info.num_subcores
    NW = NC * NS
    b_per_w = BT // NW
    CH = 128
    n_ch = b_per_w // CH
    mesh = plsc.VectorSubcoreMesh(core_axis_name="c", subcore_axis_name="s")

    @functools.partial(
        pl.kernel,
        mesh=mesh,
        out_type=jax.ShapeDtypeStruct((BT, D), jnp.float32),
        scratch_types=[
            pltpu.VMEM((CH,), jnp.int32),
            pltpu.VMEM((CH, D), jnp.float32),
            pltpu.SemaphoreType.DMA,
        ],
    )
    def gather_k(idx_hbm, cb_hbm, out_hbm, idx_v, rows_v, sem):
        wid = lax.axis_index("s") * NC + lax.axis_index("c")
        for c in range(n_ch):
            base = wid * b_per_w + c * CH
            pltpu.sync_copy(idx_hbm.at[pl.ds(base, CH)], idx_v)
            pltpu.async_copy(cb_hbm.at[idx_v], rows_v, sem).wait()
            pltpu.sync_copy(rows_v, out_hbm.at[pl.ds(base, CH)])

    return gather_k


def kernel(z_continuous, codebook):
    B, T, D = z_continuous.shape
    K = codebook.shape[0]
    BT = B * T
    z2d = z_continuous.reshape(BT, D)
    idx = _argmin_indices(z2d, codebook)
    z_q = _make_gather(BT, D, K)(idx, codebook)
    return z_q.reshape(B, T, D)
